# baseline (speedup 1.0000x reference)
import functools

import jax
import jax.numpy as jnp
from jax import lax
from jax.experimental import pallas as pl
from jax.experimental.pallas import tpu as pltpu
from jax.experimental.pallas import tpu_sc as plsc

DIM = 32
LANES = 16


def _make_sc_kernel(batch, vocab):
    info = plsc.get_sparse_core_info()
    nw = info.num_cores * info.num_subcores
    chunk = batch // nw
    n_blocks = chunk // LANES
    mesh = plsc.VectorSubcoreMesh(core_axis_name="c", subcore_axis_name="s")

    @functools.partial(
        pl.kernel,
        mesh=mesh,
        out_type=jax.ShapeDtypeStruct((batch,), jnp.float32),
        compiler_params=pltpu.CompilerParams(
            needs_layout_passes=False,
            use_tc_tiling_on_sc=False,
        ),
        scratch_types=[
            pltpu.VMEM((chunk,), jnp.int32),
            pltpu.VMEM((chunk,), jnp.int32),
            pltpu.VMEM((chunk, DIM), jnp.float32),
            pltpu.VMEM((chunk, DIM), jnp.float32),
            pltpu.VMEM((chunk,), jnp.float32),
            pltpu.VMEM((chunk,), jnp.float32),
            pltpu.VMEM((chunk,), jnp.float32),
            pltpu.SemaphoreType.DMA,
        ],
    )
    def glove_kernel(ci_hbm, xi_hbm, ctab_hbm, xtab_hbm, cb_hbm, xb_hbm,
                     out_hbm, ci_v, xi_v, crows_v, xrows_v, cb_v, xb_v,
                     out_v, sem):
        wid = lax.axis_index("s") * info.num_cores + lax.axis_index("c")
        base = pl.multiple_of(wid * chunk, chunk)

        pltpu.sync_copy(ci_hbm.at[pl.ds(base, chunk)], ci_v)
        pltpu.sync_copy(xi_hbm.at[pl.ds(base, chunk)], xi_v)

        g1 = pltpu.async_copy(ctab_hbm.at[ci_v], crows_v, sem)
        g2 = pltpu.async_copy(xtab_hbm.at[xi_v], xrows_v, sem)
        g3 = pltpu.async_copy(cb_hbm.at[ci_v], cb_v, sem)
        g4 = pltpu.async_copy(xb_hbm.at[xi_v], xb_v, sem)
        g1.wait()
        g2.wait()
        g3.wait()
        g4.wait()

        iota = lax.iota(jnp.int32, LANES)

        def blk_body(blk, carry):
            b16 = pl.multiple_of(blk * LANES, LANES)
            rows = b16 + iota
            acc = cb_v[pl.ds(b16, LANES)] + xb_v[pl.ds(b16, LANES)]
            for d in range(DIM):
                col = jnp.full((LANES,), d, jnp.int32)
                acc = acc + (plsc.load_gather(crows_v, [rows, col]) *
                             plsc.load_gather(xrows_v, [rows, col]))
            out_v[pl.ds(b16, LANES)] = acc
            return carry

        lax.fori_loop(0, n_blocks, blk_body, 0)
        pltpu.sync_copy(out_v, out_hbm.at[pl.ds(base, chunk)])

    return glove_kernel


def kernel(center_word_idx, context_word_idx, center_embeddings,
           context_embeddings, center_biases, context_biases):
    batch = center_word_idx.shape[0]
    vocab = center_embeddings.shape[0]
    ci = center_word_idx.astype(jnp.int32)
    xi = context_word_idx.astype(jnp.int32)
    cb = center_biases.reshape(vocab)
    xb = context_biases.reshape(vocab)
    sc_kernel = _make_sc_kernel(batch, vocab)
    return sc_kernel(ci, xi, center_embeddings, context_embeddings, cb, xb)
